# baseline (speedup 1.0000x reference)
import functools

import jax
import jax.numpy as jnp
from jax.experimental import pallas as pl
from jax.experimental.pallas import tpu as pltpu

TQ = 512
K = 16
TAU = 0.01
NSL = 64


def _body(q_ref, pc_ref, pcT_ref, W1_ref, b1_ref, W2_ref, b2_ref,
          Wv1_ref, bv1_ref, Wv2_ref, bv2_ref, out_ref, hpc_ref):
    qi = pl.program_id(1)
    C = W1_ref.shape[1]
    NC = pc_ref.shape[1]
    pcT = pcT_ref[0]

    @pl.when(qi == 0)
    def _():
        pc = pc_ref[0]
        h1 = jnp.maximum(
            jnp.dot(pc, W1_ref[...], preferred_element_type=jnp.float32)
            + b1_ref[...], 0.0)
        hpc_ref[:, :C] = (jnp.dot(h1, W2_ref[...],
                                  preferred_element_type=jnp.float32)
                          + b2_ref[...])
        hpc_ref[:, C:C + 3] = pc
        hpc_ref[:, C + 3:C + 4] = jnp.ones((NC, 1), jnp.float32)

    q = q_ref[0]
    c2 = 0.25 * jnp.sum(pcT * pcT, axis=0, keepdims=True)
    cross = jnp.dot(q, pcT, preferred_element_type=jnp.float32)
    d2 = c2 + cross

    pinf = jnp.float32(jnp.inf)
    NG = NC // NSL
    sls = [d2[:, a * NG:(a + 1) * NG] for a in range(NSL)]
    m1 = sls[0]
    m2 = jnp.full_like(m1, pinf)
    m3 = m2
    for s in sls[1:]:
        x = jnp.maximum(m1, s)
        m1 = jnp.minimum(m1, s)
        y = jnp.maximum(m2, x)
        m2 = jnp.minimum(m2, x)
        m3 = jnp.minimum(m3, y)
    cand = jnp.concatenate([m1, m2, m3], axis=1)

    t = jnp.full((TQ, 1), -pinf, jnp.float32)
    v1 = None
    for k in range(K):
        t = jnp.min(jnp.where(cand > t, cand, pinf), axis=1, keepdims=True)
        if k == 0:
            v1 = t

    inv_tau = jnp.float32(1.0) / jnp.float32(TAU)
    w = jnp.where(d2 <= t, jnp.exp((v1 - d2) * inv_tau), 0.0)

    agg = jnp.dot(w, hpc_ref[...], preferred_element_type=jnp.float32)
    norm = agg[:, C + 3:C + 4]
    agg_h = agg[:, :C] / norm
    rel = agg[:, C:C + 3] / norm - q

    a1 = (jnp.dot(agg_h, Wv1_ref[:C, :], preferred_element_type=jnp.float32)
          + jnp.dot(rel, Wv1_ref[C:, :], preferred_element_type=jnp.float32)
          + bv1_ref[...])
    z = jnp.maximum(a1, 0.0)
    vec = (jnp.dot(z, Wv2_ref[...], preferred_element_type=jnp.float32)
           + bv2_ref[...])
    out_ref[0] = vec


@functools.partial(jax.jit, static_argnames=())
def kernel(p_query, p_ctx, W1, b1, W2, b2, Wv1, bv1, Wv2, bv2):
    B, NQ, _ = p_query.shape
    NC = p_ctx.shape[1]
    C = W1.shape[1]
    pcT = -2.0 * jnp.swapaxes(p_ctx, 1, 2)
    grid = (B, NQ // TQ)
    zero2 = lambda b, q: (0, 0)
    out = pl.pallas_call(
        _body,
        grid=grid,
        in_specs=[
            pl.BlockSpec((1, TQ, 3), lambda b, q: (b, q, 0)),
            pl.BlockSpec((1, NC, 3), lambda b, q: (b, 0, 0)),
            pl.BlockSpec((1, 3, NC), lambda b, q: (b, 0, 0)),
            pl.BlockSpec((3, C), zero2),
            pl.BlockSpec((1, C), zero2),
            pl.BlockSpec((C, C), zero2),
            pl.BlockSpec((1, C), zero2),
            pl.BlockSpec((C + 3, C), zero2),
            pl.BlockSpec((1, C), zero2),
            pl.BlockSpec((C, 3), zero2),
            pl.BlockSpec((1, 3), zero2),
        ],
        out_specs=pl.BlockSpec((1, TQ, 3), lambda b, q: (b, q, 0)),
        out_shape=jax.ShapeDtypeStruct((B, NQ, 3), jnp.float32),
        scratch_shapes=[pltpu.VMEM((NC, C + 4), jnp.float32)],
        compiler_params=pltpu.CompilerParams(
            dimension_semantics=("parallel", "arbitrary")),
    )(p_query, p_ctx, pcT, W1, b1[None, :], W2, b2[None, :],
      Wv1, bv1[None, :], Wv2, bv2[None, :])
    return out


# speedup vs baseline: 1.0130x; 1.0130x over previous
import functools

import jax
import jax.numpy as jnp
from jax.experimental import pallas as pl
from jax.experimental.pallas import tpu as pltpu

TQ = 1024
K = 16
TAU = 0.01
NSL = 64


def _body(q_ref, pc_ref, pcT_ref, W1_ref, b1_ref, W2_ref, b2_ref,
          Wv1_ref, bv1_ref, Wv2_ref, bv2_ref, out_ref, hpc_ref):
    qi = pl.program_id(1)
    C = W1_ref.shape[1]
    NC = pc_ref.shape[1]
    pcT = pcT_ref[0]

    @pl.when(qi == 0)
    def _():
        pc = pc_ref[0]
        h1 = jnp.maximum(
            jnp.dot(pc, W1_ref[...], preferred_element_type=jnp.float32)
            + b1_ref[...], 0.0)
        hpc_ref[:, :C] = (jnp.dot(h1, W2_ref[...],
                                  preferred_element_type=jnp.float32)
                          + b2_ref[...])
        hpc_ref[:, C:C + 3] = pc
        hpc_ref[:, C + 3:C + 4] = jnp.ones((NC, 1), jnp.float32)

    q = q_ref[0]
    q2 = jnp.sum(q * q, axis=1, keepdims=True)
    c2 = 0.25 * jnp.sum(pcT * pcT, axis=0, keepdims=True)
    cross = jnp.dot(q, pcT, preferred_element_type=jnp.float32)
    d2 = (q2 + c2) + cross

    pinf = jnp.float32(jnp.inf)
    NG = NC // NSL
    sls = [d2[:, a * NG:(a + 1) * NG] for a in range(NSL)]
    m1 = sls[0]
    m2 = jnp.full_like(m1, pinf)
    m3 = m2
    for s in sls[1:]:
        x = jnp.maximum(m1, s)
        m1 = jnp.minimum(m1, s)
        y = jnp.maximum(m2, x)
        m2 = jnp.minimum(m2, x)
        m3 = jnp.minimum(m3, y)
    cand = jnp.concatenate([m1, m2, m3], axis=1)

    t = jnp.full((TQ, 1), -pinf, jnp.float32)
    v1 = None
    for k in range(K):
        t = jnp.min(jnp.where(cand > t, cand, pinf), axis=1, keepdims=True)
        if k == 0:
            v1 = t

    inv_tau = jnp.float32(1.0) / jnp.float32(TAU)
    w = jnp.where(d2 <= t, jnp.exp((v1 - d2) * inv_tau), 0.0)

    agg = jnp.dot(w, hpc_ref[...], preferred_element_type=jnp.float32)
    norm = agg[:, C + 3:C + 4]
    agg_h = agg[:, :C] / norm
    rel = agg[:, C:C + 3] / norm - q

    a1 = (jnp.dot(agg_h, Wv1_ref[:C, :], preferred_element_type=jnp.float32)
          + jnp.dot(rel, Wv1_ref[C:, :], preferred_element_type=jnp.float32)
          + bv1_ref[...])
    z = jnp.maximum(a1, 0.0)
    vec = (jnp.dot(z, Wv2_ref[...], preferred_element_type=jnp.float32)
           + bv2_ref[...])
    out_ref[0] = vec


@functools.partial(jax.jit, static_argnames=())
def kernel(p_query, p_ctx, W1, b1, W2, b2, Wv1, bv1, Wv2, bv2):
    B, NQ, _ = p_query.shape
    NC = p_ctx.shape[1]
    C = W1.shape[1]
    pcT = -2.0 * jnp.swapaxes(p_ctx, 1, 2)
    grid = (B, NQ // TQ)
    zero2 = lambda b, q: (0, 0)
    out = pl.pallas_call(
        _body,
        grid=grid,
        in_specs=[
            pl.BlockSpec((1, TQ, 3), lambda b, q: (b, q, 0)),
            pl.BlockSpec((1, NC, 3), lambda b, q: (b, 0, 0)),
            pl.BlockSpec((1, 3, NC), lambda b, q: (b, 0, 0)),
            pl.BlockSpec((3, C), zero2),
            pl.BlockSpec((1, C), zero2),
            pl.BlockSpec((C, C), zero2),
            pl.BlockSpec((1, C), zero2),
            pl.BlockSpec((C + 3, C), zero2),
            pl.BlockSpec((1, C), zero2),
            pl.BlockSpec((C, 3), zero2),
            pl.BlockSpec((1, 3), zero2),
        ],
        out_specs=pl.BlockSpec((1, TQ, 3), lambda b, q: (b, q, 0)),
        out_shape=jax.ShapeDtypeStruct((B, NQ, 3), jnp.float32),
        scratch_shapes=[pltpu.VMEM((NC, C + 4), jnp.float32)],
        compiler_params=pltpu.CompilerParams(
            dimension_semantics=("arbitrary", "arbitrary")),
    )(p_query, p_ctx, pcT, W1, b1[None, :], W2, b2[None, :],
      Wv1, bv1[None, :], Wv2, bv2[None, :])
    return out


# speedup vs baseline: 1.0346x; 1.0213x over previous
import functools

import jax
import jax.numpy as jnp
from jax.experimental import pallas as pl
from jax.experimental.pallas import tpu as pltpu

TQ = 1024
K = 16
TAU = 0.01
NSL = 64


def _body(q_ref, pc_ref, pcT_ref, W1_ref, b1_ref, W2_ref, b2_ref,
          Wv1_ref, bv1_ref, Wv2_ref, bv2_ref, out_ref, hpc_ref):
    qi = pl.program_id(1)
    C = W1_ref.shape[1]
    NC = pc_ref.shape[1]
    pcT = pcT_ref[0]

    @pl.when(qi == 0)
    def _():
        pc = pc_ref[0]
        h1 = jnp.maximum(
            jnp.dot(pc, W1_ref[...], preferred_element_type=jnp.float32)
            + b1_ref[...], 0.0)
        hpc_ref[:, :C] = (jnp.dot(h1, W2_ref[...],
                                  preferred_element_type=jnp.float32)
                          + b2_ref[...])
        hpc_ref[:, C:C + 3] = pc
        hpc_ref[:, C + 3:C + 4] = jnp.ones((NC, 1), jnp.float32)

    q = q_ref[0]
    c2 = 0.25 * jnp.sum(pcT * pcT, axis=0, keepdims=True)
    cross = jnp.dot(q, pcT, preferred_element_type=jnp.float32)
    d2 = c2 + cross

    pinf = jnp.float32(jnp.inf)
    NG = NC // NSL
    sls = [d2[:, a * NG:(a + 1) * NG] for a in range(NSL)]
    m1 = sls[0]
    m2 = jnp.full_like(m1, pinf)
    m3 = m2
    for s in sls[1:]:
        x = jnp.maximum(m1, s)
        m1 = jnp.minimum(m1, s)
        y = jnp.maximum(m2, x)
        m2 = jnp.minimum(m2, x)
        m3 = jnp.minimum(m3, y)
    cand = jnp.concatenate([m1, m2, m3], axis=1)

    t = jnp.full((TQ, 1), -pinf, jnp.float32)
    v1 = None
    for k in range(K):
        t = jnp.min(jnp.where(cand > t, cand, pinf), axis=1, keepdims=True)
        if k == 0:
            v1 = t

    inv_tau = jnp.float32(1.0) / jnp.float32(TAU)
    w = jnp.where(d2 <= t, jnp.exp((v1 - d2) * inv_tau), 0.0)

    agg = jnp.dot(w, hpc_ref[...], preferred_element_type=jnp.float32)
    norm = agg[:, C + 3:C + 4]
    agg_h = agg[:, :C] / norm
    rel = agg[:, C:C + 3] / norm - q

    a1 = (jnp.dot(agg_h, Wv1_ref[:C, :], preferred_element_type=jnp.float32)
          + jnp.dot(rel, Wv1_ref[C:, :], preferred_element_type=jnp.float32)
          + bv1_ref[...])
    z = jnp.maximum(a1, 0.0)
    vec = (jnp.dot(z, Wv2_ref[...], preferred_element_type=jnp.float32)
           + bv2_ref[...])
    out_ref[0] = vec


@functools.partial(jax.jit, static_argnames=())
def kernel(p_query, p_ctx, W1, b1, W2, b2, Wv1, bv1, Wv2, bv2):
    B, NQ, _ = p_query.shape
    NC = p_ctx.shape[1]
    C = W1.shape[1]
    pcT = -2.0 * jnp.swapaxes(p_ctx, 1, 2)
    grid = (B, NQ // TQ)
    zero2 = lambda b, q: (0, 0)
    out = pl.pallas_call(
        _body,
        grid=grid,
        in_specs=[
            pl.BlockSpec((1, TQ, 3), lambda b, q: (b, q, 0)),
            pl.BlockSpec((1, NC, 3), lambda b, q: (b, 0, 0)),
            pl.BlockSpec((1, 3, NC), lambda b, q: (b, 0, 0)),
            pl.BlockSpec((3, C), zero2),
            pl.BlockSpec((1, C), zero2),
            pl.BlockSpec((C, C), zero2),
            pl.BlockSpec((1, C), zero2),
            pl.BlockSpec((C + 3, C), zero2),
            pl.BlockSpec((1, C), zero2),
            pl.BlockSpec((C, 3), zero2),
            pl.BlockSpec((1, 3), zero2),
        ],
        out_specs=pl.BlockSpec((1, TQ, 3), lambda b, q: (b, q, 0)),
        out_shape=jax.ShapeDtypeStruct((B, NQ, 3), jnp.float32),
        scratch_shapes=[pltpu.VMEM((NC, C + 4), jnp.float32)],
        compiler_params=pltpu.CompilerParams(
            dimension_semantics=("arbitrary", "arbitrary")),
    )(p_query, p_ctx, pcT, W1, b1[None, :], W2, b2[None, :],
      Wv1, bv1[None, :], Wv2, bv2[None, :])
    return out
